# Initial kernel scaffold; baseline (speedup 1.0000x reference)
import jax
import jax.numpy as jnp
from jax.experimental import pallas as pl


def kernel(z, codebook):
    raise NotImplementedError("write your pallas kernel here")


# speedup vs baseline: 1.1635x; 1.1635x over previous
import functools

import jax
import jax.numpy as jnp
from jax import lax
from jax.experimental import pallas as pl
from jax.experimental.pallas import tpu as pltpu
from jax.experimental.pallas import tpu_sc as plsc

NUM_CODES = 1024
CODE_DIM = 64
N_TOK = 36864
BETA = 0.25

BLK = 256
GRID = N_TOK // BLK

NC = 2
NS = 16
NW = NC * NS
ROWS_PER_W = N_TOK // NW
IDX_CHUNK = 128
CHUNKS_PER_W = ROWS_PER_W // IDX_CHUNK


def _dist_argmin_body(z_ref, cbt_ref, cb_ref, idx_ref, loss_ref):
    z = z_ref[...]
    mm = jnp.dot(z, cbt_ref[...], preferred_element_type=jnp.float32)
    z_sq = jnp.sum(z * z, axis=1, keepdims=True)
    cb = cb_ref[...]
    cb_sq = jnp.sum(cb * cb, axis=1)
    d = (z_sq + cb_sq[None, :]) - 2.0 * mm
    m = jnp.min(d, axis=1, keepdims=True)
    ii = lax.broadcasted_iota(jnp.int32, (BLK, NUM_CODES), 1)
    idx = jnp.min(jnp.where(d == m, ii, NUM_CODES), axis=1)
    idx_ref[...] = idx

    @pl.when(pl.program_id(0) == 0)
    def _init():
        loss_ref[...] = jnp.zeros_like(loss_ref)

    loss_ref[...] += jnp.sum(m, axis=0, keepdims=True)


_dist_argmin_call = pl.pallas_call(
    _dist_argmin_body,
    grid=(GRID,),
    in_specs=[
        pl.BlockSpec((BLK, CODE_DIM), lambda i: (i, 0)),
        pl.BlockSpec((CODE_DIM, NUM_CODES), lambda i: (0, 0)),
        pl.BlockSpec((NUM_CODES, CODE_DIM), lambda i: (0, 0)),
    ],
    out_specs=[
        pl.BlockSpec((BLK,), lambda i: (i,)),
        pl.BlockSpec((1, 1), lambda i: (0, 0)),
    ],
    out_shape=[
        jax.ShapeDtypeStruct((N_TOK,), jnp.int32),
        jax.ShapeDtypeStruct((1, 1), jnp.float32),
    ],
)


def _sc_gather_body(cb_hbm, idx_hbm, out_hbm, idx_v, rows_v, sem):
    wid = lax.axis_index("s") * NC + lax.axis_index("c")
    pltpu.sync_copy(idx_hbm.at[pl.ds(wid * ROWS_PER_W, ROWS_PER_W)], idx_v)
    copies = [
        pltpu.async_copy(
            cb_hbm.at[idx_v.at[pl.ds(j * IDX_CHUNK, IDX_CHUNK)]],
            rows_v.at[pl.ds(j * IDX_CHUNK, IDX_CHUNK)],
            sem,
        )
        for j in range(CHUNKS_PER_W)
    ]
    for c in copies:
        c.wait()
    pltpu.sync_copy(rows_v, out_hbm.at[pl.ds(wid * ROWS_PER_W, ROWS_PER_W)])


@functools.cache
def _sc_gather():
    return pl.kernel(
        _sc_gather_body,
        out_type=jax.ShapeDtypeStruct((N_TOK, CODE_DIM), jnp.float32),
        mesh=plsc.VectorSubcoreMesh(core_axis_name="c", subcore_axis_name="s"),
        compiler_params=pltpu.CompilerParams(use_tc_tiling_on_sc=False),
        scratch_types=[
            pltpu.VMEM((ROWS_PER_W,), jnp.int32),
            pltpu.VMEM((ROWS_PER_W, CODE_DIM), jnp.float32),
            pltpu.SemaphoreType.DMA,
        ],
    )


def kernel(z, codebook):
    indices, loss_sum = _dist_argmin_call(z, codebook.T, codebook)
    z_q = _sc_gather()(codebook, indices)
    commit_loss = loss_sum[0, 0] * (BETA / (N_TOK * CODE_DIM))
    return (z_q, indices, commit_loss)


# speedup vs baseline: 1.2705x; 1.0920x over previous
import functools

import jax
import jax.numpy as jnp
from jax import lax
from jax.experimental import pallas as pl
from jax.experimental.pallas import tpu as pltpu
from jax.experimental.pallas import tpu_sc as plsc

NUM_CODES = 1024
CODE_DIM = 64
N_TOK = 36864
BETA = 0.25

BLK = 256
GRID = N_TOK // BLK

NC = 2
NS = 16
NW = NC * NS
ROWS_PER_W = N_TOK // NW
IDX_CHUNK = 128
CHUNKS_PER_W = ROWS_PER_W // IDX_CHUNK


def _dist_argmin_body(z_ref, cb_ref, idx_ref, loss_ref, cbsq_ref, iota_ref,
                      cbt2_ref):
    @pl.when(pl.program_id(0) == 0)
    def _init():
        cb = cb_ref[...]
        cbsq_ref[...] = jnp.sum(cb * cb, axis=1)[None, :]
        cbt2_ref[...] = (cb + cb).T
        iota_ref[...] = lax.broadcasted_iota(
            jnp.int32, (1, NUM_CODES), 1).astype(jnp.float32)
        loss_ref[...] = jnp.zeros_like(loss_ref)

    z = z_ref[...]
    mm2 = jnp.dot(z, cbt2_ref[...], preferred_element_type=jnp.float32)
    z_sq = jnp.sum(z * z, axis=1, keepdims=True)
    d = (z_sq + cbsq_ref[...]) - mm2
    m = jnp.min(d, axis=1, keepdims=True)
    idx = jnp.min(jnp.where(d == m, iota_ref[...], float(NUM_CODES)),
                  axis=1, keepdims=True)
    idx_ref[...] = idx.astype(jnp.int32)
    loss_ref[...] += jnp.sum(m, axis=0, keepdims=True)


_dist_argmin_call = pl.pallas_call(
    _dist_argmin_body,
    grid=(GRID,),
    in_specs=[
        pl.BlockSpec((BLK, CODE_DIM), lambda i: (i, 0)),
        pl.BlockSpec((NUM_CODES, CODE_DIM), lambda i: (0, 0)),
    ],
    out_specs=[
        pl.BlockSpec((BLK, 1), lambda i: (i, 0)),
        pl.BlockSpec((1, 1), lambda i: (0, 0)),
    ],
    out_shape=[
        jax.ShapeDtypeStruct((N_TOK, 1), jnp.int32),
        jax.ShapeDtypeStruct((1, 1), jnp.float32),
    ],
    scratch_shapes=[pltpu.VMEM((1, NUM_CODES), jnp.float32),
                    pltpu.VMEM((1, NUM_CODES), jnp.float32),
                    pltpu.VMEM((CODE_DIM, NUM_CODES), jnp.float32)],
)


def _sc_gather_body(cb_hbm, idx_hbm, out_hbm, idx_v, rows_v, sem):
    wid = lax.axis_index("s") * NC + lax.axis_index("c")
    pltpu.sync_copy(idx_hbm.at[pl.ds(wid * ROWS_PER_W, ROWS_PER_W)], idx_v)
    copies = [
        pltpu.async_copy(
            cb_hbm.at[idx_v.at[pl.ds(j * IDX_CHUNK, IDX_CHUNK)]],
            rows_v.at[pl.ds(j * IDX_CHUNK, IDX_CHUNK)],
            sem,
        )
        for j in range(CHUNKS_PER_W)
    ]
    for c in copies:
        c.wait()
    pltpu.sync_copy(rows_v, out_hbm.at[pl.ds(wid * ROWS_PER_W, ROWS_PER_W)])


@functools.cache
def _sc_gather():
    return pl.kernel(
        _sc_gather_body,
        out_type=jax.ShapeDtypeStruct((N_TOK, CODE_DIM), jnp.float32),
        mesh=plsc.VectorSubcoreMesh(core_axis_name="c", subcore_axis_name="s"),
        compiler_params=pltpu.CompilerParams(use_tc_tiling_on_sc=False),
        scratch_types=[
            pltpu.VMEM((ROWS_PER_W,), jnp.int32),
            pltpu.VMEM((ROWS_PER_W, CODE_DIM), jnp.float32),
            pltpu.SemaphoreType.DMA,
        ],
    )


def kernel(z, codebook):
    idx_col, loss_sum = _dist_argmin_call(z, codebook)
    indices = idx_col.reshape(N_TOK)
    z_q = _sc_gather()(codebook, indices)
    commit_loss = loss_sum[0, 0] * (BETA / (N_TOK * CODE_DIM))
    return (z_q, indices, commit_loss)


# speedup vs baseline: 1.3753x; 1.0825x over previous
import functools

import jax
import jax.numpy as jnp
from jax import lax
from jax.experimental import pallas as pl
from jax.experimental.pallas import tpu as pltpu
from jax.experimental.pallas import tpu_sc as plsc

NUM_CODES = 1024
CODE_DIM = 64
N_TOK = 36864
BETA = 0.25

BLK = 256
GRID = N_TOK // BLK

NC = 2
NS = 16
NW = NC * NS
ROWS_PER_W = N_TOK // NW
IDX_CHUNK = 128
CHUNKS_PER_W = ROWS_PER_W // IDX_CHUNK


def _dist_argmin_body(z_ref, cb_ref, idx_ref, loss_ref, cbsq_ref, iota_ref,
                      cbt2_ref):
    @pl.when(pl.program_id(0) == 0)
    def _init():
        cb = cb_ref[...]
        cbsq_ref[...] = jnp.sum(cb * cb, axis=1)[None, :]
        cbt2_ref[...] = (cb + cb).T
        iota_ref[...] = lax.broadcasted_iota(
            jnp.int32, (1, NUM_CODES), 1).astype(jnp.float32)
        loss_ref[...] = jnp.zeros_like(loss_ref)

    z = z_ref[...]
    mm2 = jnp.dot(z, cbt2_ref[...], preferred_element_type=jnp.float32)
    z_sq = jnp.sum(z * z, axis=1, keepdims=True)
    d = (z_sq + cbsq_ref[...]) - mm2
    m = jnp.min(d, axis=1, keepdims=True)
    idx = jnp.min(jnp.where(d == m, iota_ref[...], float(NUM_CODES)),
                  axis=1, keepdims=True)
    idx_ref[...] = idx.astype(jnp.int32).reshape(1, BLK // 128, 128)
    loss_ref[...] += jnp.sum(m, axis=0, keepdims=True)


_dist_argmin_call = pl.pallas_call(
    _dist_argmin_body,
    grid=(GRID,),
    in_specs=[
        pl.BlockSpec((BLK, CODE_DIM), lambda i: (i, 0)),
        pl.BlockSpec((NUM_CODES, CODE_DIM), lambda i: (0, 0)),
    ],
    out_specs=[
        pl.BlockSpec((1, BLK // 128, 128), lambda i: (i, 0, 0)),
        pl.BlockSpec((1, 1), lambda i: (0, 0)),
    ],
    out_shape=[
        jax.ShapeDtypeStruct((GRID, BLK // 128, 128), jnp.int32),
        jax.ShapeDtypeStruct((1, 1), jnp.float32),
    ],
    scratch_shapes=[pltpu.VMEM((1, NUM_CODES), jnp.float32),
                    pltpu.VMEM((1, NUM_CODES), jnp.float32),
                    pltpu.VMEM((CODE_DIM, NUM_CODES), jnp.float32)],
)


def _sc_gather_body(cb_hbm, idx_hbm, out_hbm, idx_v, rows_v, sem):
    wid = lax.axis_index("s") * NC + lax.axis_index("c")
    pltpu.sync_copy(idx_hbm.at[pl.ds(wid * ROWS_PER_W, ROWS_PER_W)], idx_v)
    copies = [
        pltpu.async_copy(
            cb_hbm.at[idx_v.at[pl.ds(j * IDX_CHUNK, IDX_CHUNK)]],
            rows_v.at[pl.ds(j * IDX_CHUNK, IDX_CHUNK)],
            sem,
        )
        for j in range(CHUNKS_PER_W)
    ]
    for c in copies:
        c.wait()
    pltpu.sync_copy(rows_v, out_hbm.at[pl.ds(wid * ROWS_PER_W, ROWS_PER_W)])


@functools.cache
def _sc_gather():
    return pl.kernel(
        _sc_gather_body,
        out_type=jax.ShapeDtypeStruct((N_TOK, CODE_DIM), jnp.float32),
        mesh=plsc.VectorSubcoreMesh(core_axis_name="c", subcore_axis_name="s"),
        compiler_params=pltpu.CompilerParams(use_tc_tiling_on_sc=False),
        scratch_types=[
            pltpu.VMEM((ROWS_PER_W,), jnp.int32),
            pltpu.VMEM((ROWS_PER_W, CODE_DIM), jnp.float32),
            pltpu.SemaphoreType.DMA,
        ],
    )


def kernel(z, codebook):
    idx_rows, loss_sum = _dist_argmin_call(z, codebook)
    indices = idx_rows.reshape(N_TOK)
    z_q = _sc_gather()(codebook, indices)
    commit_loss = loss_sum[0, 0] * (BETA / (N_TOK * CODE_DIM))
    return (z_q, indices, commit_loss)


# speedup vs baseline: 1.7447x; 1.2686x over previous
import functools

import jax
import jax.numpy as jnp
from jax import lax
from jax.experimental import pallas as pl
from jax.experimental.pallas import tpu as pltpu
from jax.experimental.pallas import tpu_sc as plsc

NUM_CODES = 1024
CODE_DIM = 64
N_TOK = 36864
BETA = 0.25

BLK = 512
GRID = N_TOK // BLK

NC = 2
NS = 16
NW = NC * NS
ROWS_PER_W = N_TOK // NW
IDX_CHUNK = 128
CHUNKS_PER_W = ROWS_PER_W // IDX_CHUNK


def _dist_argmin_body(z_ref, cb_ref, idx_ref, loss_ref, cbsq_ref, iota_ref,
                      cbt2_ref):
    @pl.when(pl.program_id(0) == 0)
    def _init():
        cb = cb_ref[...]
        cbsq_ref[...] = jnp.sum(cb * cb, axis=1)[None, :]
        cbt2_ref[...] = (cb + cb).T
        iota_ref[...] = lax.broadcasted_iota(
            jnp.int32, (1, NUM_CODES), 1).astype(jnp.float32)
        loss_ref[...] = jnp.zeros_like(loss_ref)

    z = z_ref[...]
    mm2 = jnp.dot(z, cbt2_ref[...], preferred_element_type=jnp.float32)
    z_sq = jnp.sum(z * z, axis=1, keepdims=True)
    d = (z_sq + cbsq_ref[...]) - mm2
    m = jnp.min(d, axis=1, keepdims=True)
    idx = jnp.min(jnp.where(d == m, iota_ref[...], float(NUM_CODES)),
                  axis=1, keepdims=True)
    idx_ref[...] = idx.astype(jnp.int32).reshape(1, BLK // 128, 128)
    loss_ref[...] += jnp.sum(m, axis=0, keepdims=True)


_dist_argmin_call = pl.pallas_call(
    _dist_argmin_body,
    grid=(GRID,),
    in_specs=[
        pl.BlockSpec((BLK, CODE_DIM), lambda i: (i, 0)),
        pl.BlockSpec((NUM_CODES, CODE_DIM), lambda i: (0, 0)),
    ],
    out_specs=[
        pl.BlockSpec((1, BLK // 128, 128), lambda i: (i, 0, 0)),
        pl.BlockSpec((1, 1), lambda i: (0, 0)),
    ],
    out_shape=[
        jax.ShapeDtypeStruct((GRID, BLK // 128, 128), jnp.int32),
        jax.ShapeDtypeStruct((1, 1), jnp.float32),
    ],
    scratch_shapes=[pltpu.VMEM((1, NUM_CODES), jnp.float32),
                    pltpu.VMEM((1, NUM_CODES), jnp.float32),
                    pltpu.VMEM((CODE_DIM, NUM_CODES), jnp.float32)],
)


def _sc_gather_body(cb_hbm, idx_hbm, out_hbm, idx_v, rows_v, sem):
    wid = lax.axis_index("s") * NC + lax.axis_index("c")
    pltpu.sync_copy(idx_hbm.at[pl.ds(wid * ROWS_PER_W, ROWS_PER_W)], idx_v)
    copies = [
        pltpu.async_copy(
            cb_hbm.at[idx_v.at[pl.ds(j * IDX_CHUNK, IDX_CHUNK)]],
            rows_v.at[pl.ds(j * IDX_CHUNK, IDX_CHUNK)],
            sem,
        )
        for j in range(CHUNKS_PER_W)
    ]
    for c in copies:
        c.wait()
    pltpu.sync_copy(rows_v, out_hbm.at[pl.ds(wid * ROWS_PER_W, ROWS_PER_W)])


@functools.cache
def _sc_gather():
    return pl.kernel(
        _sc_gather_body,
        out_type=jax.ShapeDtypeStruct((N_TOK, CODE_DIM), jnp.float32),
        mesh=plsc.VectorSubcoreMesh(core_axis_name="c", subcore_axis_name="s"),
        compiler_params=pltpu.CompilerParams(use_tc_tiling_on_sc=False),
        scratch_types=[
            pltpu.VMEM((ROWS_PER_W,), jnp.int32),
            pltpu.VMEM((ROWS_PER_W, CODE_DIM), jnp.float32),
            pltpu.SemaphoreType.DMA,
        ],
    )


def kernel(z, codebook):
    idx_rows, loss_sum = _dist_argmin_call(z, codebook)
    indices = idx_rows.reshape(N_TOK)
    z_q = _sc_gather()(codebook, indices)
    commit_loss = loss_sum[0, 0] * (BETA / (N_TOK * CODE_DIM))
    return (z_q, indices, commit_loss)


# speedup vs baseline: 1.9294x; 1.1059x over previous
import functools

import jax
import jax.numpy as jnp
from jax import lax
from jax.experimental import pallas as pl
from jax.experimental.pallas import tpu as pltpu
from jax.experimental.pallas import tpu_sc as plsc

NUM_CODES = 1024
CODE_DIM = 64
N_TOK = 36864
BETA = 0.25

BLK = 1024
GRID = N_TOK // BLK

NC = 2
NS = 16
NW = NC * NS
ROWS_PER_W = N_TOK // NW
IDX_CHUNK = 128
CHUNKS_PER_W = ROWS_PER_W // IDX_CHUNK


def _dist_argmin_body(z_ref, cb_ref, idx_ref, loss_ref, cbsq_ref, iota_ref,
                      cbt2_ref):
    @pl.when(pl.program_id(0) == 0)
    def _init():
        cb = cb_ref[...]
        cbsq_ref[...] = jnp.sum(cb * cb, axis=1)[None, :]
        cbt2_ref[...] = (cb + cb).T
        iota_ref[...] = lax.broadcasted_iota(
            jnp.int32, (1, NUM_CODES), 1).astype(jnp.float32)
        loss_ref[...] = jnp.zeros_like(loss_ref)

    z = z_ref[...]
    mm2 = jnp.dot(z, cbt2_ref[...], preferred_element_type=jnp.float32)
    z_sq = jnp.sum(z * z, axis=1, keepdims=True)
    d = (z_sq + cbsq_ref[...]) - mm2
    m = jnp.min(d, axis=1, keepdims=True)
    idx = jnp.min(jnp.where(d == m, iota_ref[...], float(NUM_CODES)),
                  axis=1, keepdims=True)
    idx_ref[...] = idx.astype(jnp.int32).reshape(1, BLK // 128, 128)
    loss_ref[...] += jnp.sum(m, axis=0, keepdims=True)


_dist_argmin_call = pl.pallas_call(
    _dist_argmin_body,
    grid=(GRID,),
    in_specs=[
        pl.BlockSpec((BLK, CODE_DIM), lambda i: (i, 0)),
        pl.BlockSpec((NUM_CODES, CODE_DIM), lambda i: (0, 0)),
    ],
    out_specs=[
        pl.BlockSpec((1, BLK // 128, 128), lambda i: (i, 0, 0)),
        pl.BlockSpec((1, 1), lambda i: (0, 0)),
    ],
    out_shape=[
        jax.ShapeDtypeStruct((GRID, BLK // 128, 128), jnp.int32),
        jax.ShapeDtypeStruct((1, 1), jnp.float32),
    ],
    scratch_shapes=[pltpu.VMEM((1, NUM_CODES), jnp.float32),
                    pltpu.VMEM((1, NUM_CODES), jnp.float32),
                    pltpu.VMEM((CODE_DIM, NUM_CODES), jnp.float32)],
)


def _sc_gather_body(cb_hbm, idx_hbm, out_hbm, idx_v, rows_v, sem):
    wid = lax.axis_index("s") * NC + lax.axis_index("c")
    pltpu.sync_copy(idx_hbm.at[pl.ds(wid * ROWS_PER_W, ROWS_PER_W)], idx_v)
    copies = [
        pltpu.async_copy(
            cb_hbm.at[idx_v.at[pl.ds(j * IDX_CHUNK, IDX_CHUNK)]],
            rows_v.at[pl.ds(j * IDX_CHUNK, IDX_CHUNK)],
            sem,
        )
        for j in range(CHUNKS_PER_W)
    ]
    for c in copies:
        c.wait()
    pltpu.sync_copy(rows_v, out_hbm.at[pl.ds(wid * ROWS_PER_W, ROWS_PER_W)])


@functools.cache
def _sc_gather():
    return pl.kernel(
        _sc_gather_body,
        out_type=jax.ShapeDtypeStruct((N_TOK, CODE_DIM), jnp.float32),
        mesh=plsc.VectorSubcoreMesh(core_axis_name="c", subcore_axis_name="s"),
        compiler_params=pltpu.CompilerParams(use_tc_tiling_on_sc=False),
        scratch_types=[
            pltpu.VMEM((ROWS_PER_W,), jnp.int32),
            pltpu.VMEM((ROWS_PER_W, CODE_DIM), jnp.float32),
            pltpu.SemaphoreType.DMA,
        ],
    )


def kernel(z, codebook):
    idx_rows, loss_sum = _dist_argmin_call(z, codebook)
    indices = idx_rows.reshape(N_TOK)
    z_q = _sc_gather()(codebook, indices)
    commit_loss = loss_sum[0, 0] * (BETA / (N_TOK * CODE_DIM))
    return (z_q, indices, commit_loss)


# speedup vs baseline: 1.9665x; 1.0192x over previous
import functools

import jax
import jax.numpy as jnp
from jax import lax
from jax.experimental import pallas as pl
from jax.experimental.pallas import tpu as pltpu
from jax.experimental.pallas import tpu_sc as plsc

NUM_CODES = 1024
CODE_DIM = 64
N_TOK = 36864
BETA = 0.25

BLK = 2304
GRID = N_TOK // BLK

NC = 2
NS = 16
NW = NC * NS
ROWS_PER_W = N_TOK // NW
IDX_CHUNK = 128
CHUNKS_PER_W = ROWS_PER_W // IDX_CHUNK


def _dist_argmin_body(z_ref, cb_ref, idx_ref, loss_ref, cbsq_ref, iota_ref,
                      cbt2_ref):
    @pl.when(pl.program_id(0) == 0)
    def _init():
        cb = cb_ref[...]
        cbsq_ref[...] = jnp.sum(cb * cb, axis=1)[None, :]
        cbt2_ref[...] = (cb + cb).T
        iota_ref[...] = lax.broadcasted_iota(
            jnp.int32, (1, NUM_CODES), 1).astype(jnp.float32)
        loss_ref[...] = jnp.zeros_like(loss_ref)

    z = z_ref[...]
    mm2 = jnp.dot(z, cbt2_ref[...], preferred_element_type=jnp.float32)
    z_sq = jnp.sum(z * z, axis=1, keepdims=True)
    d = (z_sq + cbsq_ref[...]) - mm2
    m = jnp.min(d, axis=1, keepdims=True)
    idx = jnp.min(jnp.where(d == m, iota_ref[...], float(NUM_CODES)),
                  axis=1, keepdims=True)
    idx_ref[...] = idx.astype(jnp.int32).reshape(1, BLK // 128, 128)
    loss_ref[...] += jnp.sum(m, axis=0, keepdims=True)


_dist_argmin_call = pl.pallas_call(
    _dist_argmin_body,
    grid=(GRID,),
    in_specs=[
        pl.BlockSpec((BLK, CODE_DIM), lambda i: (i, 0)),
        pl.BlockSpec((NUM_CODES, CODE_DIM), lambda i: (0, 0)),
    ],
    out_specs=[
        pl.BlockSpec((1, BLK // 128, 128), lambda i: (i, 0, 0)),
        pl.BlockSpec((1, 1), lambda i: (0, 0)),
    ],
    out_shape=[
        jax.ShapeDtypeStruct((GRID, BLK // 128, 128), jnp.int32),
        jax.ShapeDtypeStruct((1, 1), jnp.float32),
    ],
    scratch_shapes=[pltpu.VMEM((1, NUM_CODES), jnp.float32),
                    pltpu.VMEM((1, NUM_CODES), jnp.float32),
                    pltpu.VMEM((CODE_DIM, NUM_CODES), jnp.float32)],
)


def _sc_gather_body(cb_hbm, idx_hbm, out_hbm, idx_v, rows_v, sem):
    wid = lax.axis_index("s") * NC + lax.axis_index("c")
    pltpu.sync_copy(idx_hbm.at[pl.ds(wid * ROWS_PER_W, ROWS_PER_W)], idx_v)
    copies = [
        pltpu.async_copy(
            cb_hbm.at[idx_v.at[pl.ds(j * IDX_CHUNK, IDX_CHUNK)]],
            rows_v.at[pl.ds(j * IDX_CHUNK, IDX_CHUNK)],
            sem,
        )
        for j in range(CHUNKS_PER_W)
    ]
    for c in copies:
        c.wait()
    pltpu.sync_copy(rows_v, out_hbm.at[pl.ds(wid * ROWS_PER_W, ROWS_PER_W)])


@functools.cache
def _sc_gather():
    return pl.kernel(
        _sc_gather_body,
        out_type=jax.ShapeDtypeStruct((N_TOK, CODE_DIM), jnp.float32),
        mesh=plsc.VectorSubcoreMesh(core_axis_name="c", subcore_axis_name="s"),
        compiler_params=pltpu.CompilerParams(use_tc_tiling_on_sc=False),
        scratch_types=[
            pltpu.VMEM((ROWS_PER_W,), jnp.int32),
            pltpu.VMEM((ROWS_PER_W, CODE_DIM), jnp.float32),
            pltpu.SemaphoreType.DMA,
        ],
    )


def kernel(z, codebook):
    idx_rows, loss_sum = _dist_argmin_call(z, codebook)
    indices = idx_rows.reshape(N_TOK)
    z_q = _sc_gather()(codebook, indices)
    commit_loss = loss_sum[0, 0] * (BETA / (N_TOK * CODE_DIM))
    return (z_q, indices, commit_loss)


# speedup vs baseline: 2.1569x; 1.0968x over previous
import functools

import jax
import jax.numpy as jnp
from jax import lax
from jax.experimental import pallas as pl
from jax.experimental.pallas import tpu as pltpu
from jax.experimental.pallas import tpu_sc as plsc

NUM_CODES = 1024
CODE_DIM = 64
N_TOK = 36864
BETA = 0.25

BLK = 2304
GRID = N_TOK // BLK

NC = 2
NS = 16
NW = NC * NS
ROWS_PER_W = N_TOK // NW
IDX_CHUNK = 128
CHUNKS_PER_W = ROWS_PER_W // IDX_CHUNK


def _dist_argmin_body(z_ref, cb_ref, idx_ref, loss_ref, cbsq_ref, iota_ref,
                      cbt2_ref):
    @pl.when(pl.program_id(0) == 0)
    def _init():
        cb = cb_ref[...]
        cbsq_ref[...] = jnp.sum(cb * cb, axis=1)[None, :]
        cbt2_ref[...] = (cb + cb).T
        iota_ref[...] = lax.broadcasted_iota(
            jnp.int32, (1, NUM_CODES), 1).astype(jnp.float32)
        loss_ref[...] = jnp.zeros_like(loss_ref)

    z = z_ref[...]
    mm2 = jnp.dot(z, cbt2_ref[...], preferred_element_type=jnp.float32)
    z_sq = jnp.sum(z * z, axis=1, keepdims=True)
    LW = 128
    run_min = None
    for c in range(NUM_CODES // LW):
        cs = slice(c * LW, (c + 1) * LW)
        dc = (z_sq + cbsq_ref[...][:, cs]) - mm2[:, cs]
        ic = iota_ref[...][:, cs]
        if run_min is None:
            run_min, run_idx = dc, jnp.broadcast_to(ic, dc.shape)
        else:
            better = dc < run_min
            run_idx = jnp.where(better, ic, run_idx)
            run_min = jnp.minimum(run_min, dc)
    m = jnp.min(run_min, axis=1, keepdims=True)
    idx = jnp.min(jnp.where(run_min == m, run_idx, float(NUM_CODES)),
                  axis=1, keepdims=True)
    idx_ref[...] = idx.astype(jnp.int32).reshape(1, BLK // 128, 128)
    loss_ref[...] += jnp.sum(m, axis=0, keepdims=True)


_dist_argmin_call = pl.pallas_call(
    _dist_argmin_body,
    grid=(GRID,),
    in_specs=[
        pl.BlockSpec((BLK, CODE_DIM), lambda i: (i, 0)),
        pl.BlockSpec((NUM_CODES, CODE_DIM), lambda i: (0, 0)),
    ],
    out_specs=[
        pl.BlockSpec((1, BLK // 128, 128), lambda i: (i, 0, 0)),
        pl.BlockSpec((1, 1), lambda i: (0, 0)),
    ],
    out_shape=[
        jax.ShapeDtypeStruct((GRID, BLK // 128, 128), jnp.int32),
        jax.ShapeDtypeStruct((1, 1), jnp.float32),
    ],
    scratch_shapes=[pltpu.VMEM((1, NUM_CODES), jnp.float32),
                    pltpu.VMEM((1, NUM_CODES), jnp.float32),
                    pltpu.VMEM((CODE_DIM, NUM_CODES), jnp.float32)],
)


def _sc_gather_body(cb_hbm, idx_hbm, out_hbm, idx_v, rows_v, sem):
    wid = lax.axis_index("s") * NC + lax.axis_index("c")
    pltpu.sync_copy(idx_hbm.at[pl.ds(wid * ROWS_PER_W, ROWS_PER_W)], idx_v)
    copies = [
        pltpu.async_copy(
            cb_hbm.at[idx_v.at[pl.ds(j * IDX_CHUNK, IDX_CHUNK)]],
            rows_v.at[pl.ds(j * IDX_CHUNK, IDX_CHUNK)],
            sem,
        )
        for j in range(CHUNKS_PER_W)
    ]
    for c in copies:
        c.wait()
    pltpu.sync_copy(rows_v, out_hbm.at[pl.ds(wid * ROWS_PER_W, ROWS_PER_W)])


@functools.cache
def _sc_gather():
    return pl.kernel(
        _sc_gather_body,
        out_type=jax.ShapeDtypeStruct((N_TOK, CODE_DIM), jnp.float32),
        mesh=plsc.VectorSubcoreMesh(core_axis_name="c", subcore_axis_name="s"),
        compiler_params=pltpu.CompilerParams(use_tc_tiling_on_sc=False),
        scratch_types=[
            pltpu.VMEM((ROWS_PER_W,), jnp.int32),
            pltpu.VMEM((ROWS_PER_W, CODE_DIM), jnp.float32),
            pltpu.SemaphoreType.DMA,
        ],
    )


def kernel(z, codebook):
    idx_rows, loss_sum = _dist_argmin_call(z, codebook)
    indices = idx_rows.reshape(N_TOK)
    z_q = _sc_gather()(codebook, indices)
    commit_loss = loss_sum[0, 0] * (BETA / (N_TOK * CODE_DIM))
    return (z_q, indices, commit_loss)
